# baseline (speedup 1.0000x reference)
import dataclasses
import functools

import jax
import jax.numpy as jnp
from jax import lax
from jax.experimental import pallas as pl
from jax.experimental.pallas import tpu as pltpu
from jax.experimental.pallas import tpu_sc as plsc

NC, NS = 2, 16
NW = NC * NS
WIN = 80


def _sc_mesh():
    return plsc.VectorSubcoreMesh(core_axis_name="c", subcore_axis_name="s",
                                  num_cores=NC, num_subcores=NS)


def _sc_params():
    cp = pltpu.CompilerParams()
    if "needs_layout_passes" in pltpu.CompilerParams.__dataclass_fields__:
        cp = dataclasses.replace(cp, needs_layout_passes=False)
    return cp


def _make_deg(E, n_pad):
    per_w = E // NW
    n_win = per_w // WIN
    rps = n_pad // NS

    @functools.partial(
        pl.kernel,
        out_type=jax.ShapeDtypeStruct((NC, n_pad), jnp.float32),
        mesh=_sc_mesh(),
        scratch_types=[
            [pltpu.VMEM((WIN,), jnp.int32) for _ in range(4)],
            pltpu.VMEM((n_pad,), jnp.float32),
            pltpu.VMEM((NS, rps), jnp.float32),
            pltpu.VMEM_SHARED((NS, n_pad), jnp.float32),
            [pltpu.SemaphoreType.DMA for _ in range(4)],
        ],
        compiler_params=_sc_params(),
    )
    def deg(dst_hbm, out_hbm, di, hist, red, stage_sh, semi):
        cid = lax.axis_index("c")
        sid = lax.axis_index("s")
        wid = sid * NC + cid
        base0 = wid * per_w
        ones16 = jnp.ones((16,), jnp.float32)

        @pl.loop(0, n_pad, step=16)
        def _(i):
            hist[pl.ds(i, 16)] = jnp.zeros((16,), jnp.float32)

        def issue_idx(t, q):
            pltpu.async_copy(dst_hbm.at[pl.ds(base0 + t * WIN, WIN)], di[q],
                             semi[q])

        def window(t, q):
            @pl.when(t + 2 < n_win)
            def _():
                issue_idx(t + 2, (q + 2) % 4)
            pltpu.make_async_copy(dst_hbm.at[pl.ds(base0 + t * WIN, WIN)],
                                  di[q], semi[q]).wait()
            for c in range(WIN // 16):
                idx = di[q][pl.ds(c * 16, 16)]
                plsc.addupdate_scatter(hist, [idx], ones16)

        issue_idx(0, 0)
        issue_idx(1, 1)

        @pl.loop(0, n_win - 1, step=4)
        def _(t):
            window(t, 0)
            window(t + 1, 1)
            window(t + 2, 2)
            window(t + 3, 3)

        window(n_win - 1, 0)

        pltpu.sync_copy(hist, stage_sh.at[sid])
        plsc.subcore_barrier()
        pltpu.sync_copy(stage_sh.at[:, pl.ds(sid * rps, rps)], red)

        @pl.loop(0, rps, step=16)
        def _(i):
            acc16 = red[0, pl.ds(i, 16)]
            for r in range(1, NS):
                acc16 = acc16 + red[r, pl.ds(i, 16)]
            red[0, pl.ds(i, 16)] = acc16

        pltpu.sync_copy(red.at[0], out_hbm.at[cid, pl.ds(sid * rps, rps)])

    return deg


def _make_agg(E, n_pad, hid, n_rows):
    per_w = E // NW
    n_win = per_w // WIN
    rps = n_pad // NS

    n_loop = ((n_win + 7) // 8) * 8
    n_fs = n_rows // rps
    rem = n_rows - n_fs * rps

    @functools.partial(
        pl.kernel,
        out_type=jax.ShapeDtypeStruct((NC, n_pad, hid), jnp.float32),
        mesh=_sc_mesh(),
        scratch_types=[
            [pltpu.VMEM((WIN,), jnp.int32) for _ in range(8)],
            [pltpu.VMEM((WIN,), jnp.int32) for _ in range(8)],
            [pltpu.VMEM((WIN, hid), jnp.float32) for _ in range(4)],
            pltpu.VMEM_SHARED((n_pad, hid), jnp.float32),
            [pltpu.SemaphoreType.DMA for _ in range(8)],
            [pltpu.SemaphoreType.DMA for _ in range(4)],
            [pltpu.SemaphoreType.DMA for _ in range(4)],
        ],
    )
    def agg(m_hbm, src_hbm, dst_hbm, out_hbm, si, di, rows, acc_sh, semi,
            semg, sems):
        cid = lax.axis_index("c")
        sid = lax.axis_index("s")
        wid = sid * NC + cid
        base0 = wid * per_w

        @pl.when(cid == 0)
        def _():
            @pl.when(sid < n_fs)
            def _():
                pltpu.sync_copy(m_hbm.at[pl.ds(sid * rps, rps)],
                                acc_sh.at[pl.ds(sid * rps, rps)])
            if rem:
                @pl.when(sid == n_fs)
                def _():
                    pltpu.sync_copy(m_hbm.at[pl.ds(n_fs * rps, rem)],
                                    acc_sh.at[pl.ds(n_fs * rps, rem)])

        @pl.when(cid == 1)
        def _():
            @pl.loop(0, WIN)
            def _(r):
                @pl.loop(0, hid, step=16)
                def _(k):
                    rows[0][r, pl.ds(k, 16)] = jnp.zeros((16,), jnp.float32)

            @pl.loop(0, rps // WIN)
            def _(t):
                pltpu.sync_copy(rows[0],
                                acc_sh.at[pl.ds(sid * rps + t * WIN, WIN)])

        plsc.subcore_barrier()

        def issue_idx(t, q):
            sl = pl.ds(base0 + t * WIN, WIN)
            pltpu.async_copy(src_hbm.at[sl], si[q], semi[q])
            pltpu.async_copy(dst_hbm.at[sl], di[q], semi[q])

        def wait_idx(t, q):
            sl = pl.ds(base0 + t * WIN, WIN)
            pltpu.make_async_copy(src_hbm.at[sl], si[q], semi[q]).wait()
            pltpu.make_async_copy(dst_hbm.at[sl], di[q], semi[q]).wait()

        def drain_scatter(k):
            pltpu.make_async_copy(rows[k % 4], acc_sh.at[di[k % 8]],
                                  sems[k % 4]).wait()

        def stage(tb, k):
            t = tb + k

            @pl.when(t >= 4)
            def _():
                drain_scatter(k - 4)

            @pl.when(t + 4 < n_win)
            def _():
                issue_idx(t + 4, (k + 4) % 8)

            @pl.when(t < n_win)
            def _():
                wait_idx(t, k)
                pltpu.async_copy(m_hbm.at[si[k]], rows[k % 4], semg[k % 4])

            @pl.when(jnp.logical_and(t >= 2, t - 2 < n_win))
            def _():
                pltpu.make_async_copy(m_hbm.at[si[(k - 2) % 8]],
                                      rows[(k - 2) % 4],
                                      semg[(k - 2) % 4]).wait()
                pltpu.async_copy(rows[(k - 2) % 4],
                                 acc_sh.at[di[(k - 2) % 8]],
                                 sems[(k - 2) % 4], add=True)

        for t0 in range(4):
            issue_idx(t0, t0)

        @pl.loop(0, n_loop, step=8)
        def _(tb):
            for k in range(8):
                stage(tb, k)

        drain_scatter(n_win - 1)

        plsc.subcore_barrier()
        pltpu.sync_copy(acc_sh.at[pl.ds(sid * rps, rps)],
                        out_hbm.at[cid, pl.ds(sid * rps, rps)])

    return agg


def _split_body(e_ref, s_ref, d_ref):
    v = e_ref[...]
    s_ref[...] = v[0]
    d_ref[...] = v[1]


def _xws_body(x_ref, w_ref, cnt_ref, m_ref, s_ref):
    cb = cnt_ref[...]
    s_row = lax.rsqrt(cb[0:1] + cb[1:2] + 1.0)
    s_col = jnp.transpose(s_row, (1, 0))
    s_ref[...] = s_col
    xw = jnp.dot(x_ref[...], w_ref[...],
                 preferred_element_type=jnp.float32)
    m_ref[...] = xw * s_col


def _gru_body(acc_ref, s_ref, bgcn_ref, wih_ref, bih_ref, bhh_ref,
              wfc_ref, bfc_ref, o_ref, *, hid):
    acc2 = acc_ref[...]
    g = s_ref[...] * (acc2[0] + acc2[1]) + bgcn_ref[...]
    gi = jnp.dot(g.astype(jnp.bfloat16), wih_ref[...],
                 preferred_element_type=jnp.float32)
    gi = gi + bih_ref[...]
    bhh = bhh_ref[...]
    r = jax.nn.sigmoid(gi[:, :hid] + bhh[:, :hid])
    z = jax.nn.sigmoid(gi[:, hid:2 * hid] + bhh[:, hid:2 * hid])
    nn_ = jnp.tanh(gi[:, 2 * hid:] + r * bhh[:, 2 * hid:])
    h = (1.0 - z) * nn_
    o_ref[...] = jnp.dot(h.astype(jnp.bfloat16), wfc_ref[...],
                         preferred_element_type=jnp.float32) + bfc_ref[...]


def kernel(x_seq, edge_idx, W_gcn, b_gcn, W_ih, b_ih, W_hh, b_hh, W_fc, b_fc):
    n, t_in = x_seq.shape
    hid = W_gcn.shape[1]
    e = edge_idx.shape[1]
    n_pad = ((n + NS * WIN - 1) // (NS * WIN)) * (NS * WIN)
    blk = 1000
    grid = (n // blk,)

    blk_e = e
    src, dst = pl.pallas_call(
        _split_body,
        grid=(e // blk_e,),
        in_specs=[pl.BlockSpec((2, blk_e), lambda i: (0, i))],
        out_specs=[pl.BlockSpec((blk_e,), lambda i: (i,)),
                   pl.BlockSpec((blk_e,), lambda i: (i,))],
        out_shape=[jax.ShapeDtypeStruct((e,), jnp.int32),
                   jax.ShapeDtypeStruct((e,), jnp.int32)],
    )(edge_idx)

    cnt = _make_deg(e, n_pad)(dst)

    blk_x = 2560
    m, s_col = pl.pallas_call(
        _xws_body,
        grid=(n_pad // blk_x,),
        in_specs=[pl.BlockSpec((blk_x, t_in), lambda i: (i, 0)),
                  pl.BlockSpec((t_in, hid), lambda i: (0, 0)),
                  pl.BlockSpec((NC, blk_x), lambda i: (0, i))],
        out_specs=[pl.BlockSpec((blk_x, hid), lambda i: (i, 0)),
                   pl.BlockSpec((blk_x, 1), lambda i: (i, 0))],
        out_shape=[jax.ShapeDtypeStruct((n, hid), jnp.float32),
                   jax.ShapeDtypeStruct((n, 1), jnp.float32)],
    )(x_seq, W_gcn, cnt)

    acc = _make_agg(e, n_pad, hid, n)(m, src, dst)

    blk_g = 2000
    out2 = pl.pallas_call(
        functools.partial(_gru_body, hid=hid),
        grid=(n // blk_g,),
        in_specs=[
            pl.BlockSpec((NC, blk_g, hid), lambda i: (0, i, 0)),
            pl.BlockSpec((blk_g, 1), lambda i: (i, 0)),
            pl.BlockSpec((1, hid), lambda i: (0, 0)),
            pl.BlockSpec((hid, 3 * hid), lambda i: (0, 0)),
            pl.BlockSpec((1, 3 * hid), lambda i: (0, 0)),
            pl.BlockSpec((1, 3 * hid), lambda i: (0, 0)),
            pl.BlockSpec((hid, 1), lambda i: (0, 0)),
            pl.BlockSpec((1, 1), lambda i: (0, 0)),
        ],
        out_specs=pl.BlockSpec((blk_g, 1), lambda i: (i, 0)),
        out_shape=jax.ShapeDtypeStruct((n, 1), jnp.float32),
    )(acc, s_col, b_gcn[None, :], W_ih.T.astype(jnp.bfloat16),
      b_ih[None, :], b_hh[None, :], W_fc.T.astype(jnp.bfloat16),
      b_fc[None, :])

    return out2[:, 0]


# speedup vs baseline: 1.0304x; 1.0304x over previous
import functools

import jax
import jax.numpy as jnp
from jax import lax
from jax.experimental import pallas as pl
from jax.experimental.pallas import tpu as pltpu
from jax.experimental.pallas import tpu_sc as plsc

NC, NS = 2, 16
NW = NC * NS
WIN = 80


def _sc_mesh():
    return plsc.VectorSubcoreMesh(core_axis_name="c", subcore_axis_name="s",
                                  num_cores=NC, num_subcores=NS)

def _make_deg(E, n_pad):
    per_w = E // NW
    n_win = per_w // WIN
    rps = n_pad // NS

    @functools.partial(
        pl.kernel,
        out_type=jax.ShapeDtypeStruct((NC, n_pad), jnp.float32),
        mesh=_sc_mesh(),
        scratch_types=[
            [pltpu.VMEM((WIN,), jnp.int32) for _ in range(4)],
            pltpu.VMEM((WIN,), jnp.float32),
            pltpu.VMEM((rps,), jnp.float32),
            pltpu.VMEM_SHARED((n_pad,), jnp.float32),
            [pltpu.SemaphoreType.DMA for _ in range(4)],
            [pltpu.SemaphoreType.DMA for _ in range(2)],
        ],
    )
    def deg(dst_hbm, out_hbm, di, ones_v, z_v, deg_sh, semi, semd):
        cid = lax.axis_index("c")
        sid = lax.axis_index("s")
        wid = sid * NC + cid
        base0 = wid * per_w

        @pl.loop(0, WIN, step=16)
        def _(i):
            ones_v[pl.ds(i, 16)] = jnp.ones((16,), jnp.float32)

        @pl.loop(0, rps, step=16)
        def _(i):
            z_v[pl.ds(i, 16)] = jnp.zeros((16,), jnp.float32)

        pltpu.sync_copy(z_v, deg_sh.at[pl.ds(sid * rps, rps)])
        plsc.subcore_barrier()

        def issue_idx(t, q):
            pltpu.async_copy(dst_hbm.at[pl.ds(base0 + t * WIN, WIN)], di[q],
                             semi[q])

        def window(t, q, p):
            @pl.when(t >= 2)
            def _():
                pltpu.make_async_copy(ones_v, deg_sh.at[di[q]],
                                      semd[p]).wait()
            @pl.when(t + 2 < n_win)
            def _():
                issue_idx(t + 2, (q + 2) % 4)
            pltpu.make_async_copy(dst_hbm.at[pl.ds(base0 + t * WIN, WIN)],
                                  di[q], semi[q]).wait()
            pltpu.async_copy(ones_v, deg_sh.at[di[q]], semd[p], add=True)

        issue_idx(0, 0)
        issue_idx(1, 1)

        @pl.loop(0, n_win - 1, step=4)
        def _(t):
            window(t, 0, 0)
            window(t + 1, 1, 1)
            window(t + 2, 2, 0)
            window(t + 3, 3, 1)

        window(n_win - 1, 0, 0)
        pltpu.make_async_copy(ones_v, deg_sh.at[di[0]], semd[0]).wait()
        pltpu.make_async_copy(ones_v, deg_sh.at[di[1]], semd[1]).wait()

        plsc.subcore_barrier()
        pltpu.sync_copy(deg_sh.at[pl.ds(sid * rps, rps)],
                        out_hbm.at[cid, pl.ds(sid * rps, rps)])

    return deg


def _make_agg(E, n_pad, hid, n_rows):
    per_w = E // NW
    n_win = per_w // WIN
    rps = n_pad // NS

    n_loop = ((n_win + 7) // 8) * 8
    n_fs = n_rows // rps
    rem = n_rows - n_fs * rps

    @functools.partial(
        pl.kernel,
        out_type=jax.ShapeDtypeStruct((NC, n_pad, hid), jnp.float32),
        mesh=_sc_mesh(),
        scratch_types=[
            [pltpu.VMEM((WIN,), jnp.int32) for _ in range(8)],
            [pltpu.VMEM((WIN,), jnp.int32) for _ in range(8)],
            [pltpu.VMEM((WIN, hid), jnp.float32) for _ in range(4)],
            pltpu.VMEM_SHARED((n_pad, hid), jnp.float32),
            [pltpu.SemaphoreType.DMA for _ in range(8)],
            [pltpu.SemaphoreType.DMA for _ in range(4)],
            [pltpu.SemaphoreType.DMA for _ in range(4)],
        ],
    )
    def agg(m_hbm, src_hbm, dst_hbm, out_hbm, si, di, rows, acc_sh, semi,
            semg, sems):
        cid = lax.axis_index("c")
        sid = lax.axis_index("s")
        wid = sid * NC + cid
        base0 = wid * per_w

        @pl.when(cid == 0)
        def _():
            @pl.when(sid < n_fs)
            def _():
                pltpu.sync_copy(m_hbm.at[pl.ds(sid * rps, rps)],
                                acc_sh.at[pl.ds(sid * rps, rps)])
            if rem:
                @pl.when(sid == n_fs)
                def _():
                    pltpu.sync_copy(m_hbm.at[pl.ds(n_fs * rps, rem)],
                                    acc_sh.at[pl.ds(n_fs * rps, rem)])

        @pl.when(cid == 1)
        def _():
            @pl.loop(0, WIN)
            def _(r):
                @pl.loop(0, hid, step=16)
                def _(k):
                    rows[0][r, pl.ds(k, 16)] = jnp.zeros((16,), jnp.float32)

            @pl.loop(0, rps // WIN)
            def _(t):
                pltpu.sync_copy(rows[0],
                                acc_sh.at[pl.ds(sid * rps + t * WIN, WIN)])

        plsc.subcore_barrier()

        def issue_idx(t, q):
            sl = pl.ds(base0 + t * WIN, WIN)
            pltpu.async_copy(src_hbm.at[sl], si[q], semi[q])
            pltpu.async_copy(dst_hbm.at[sl], di[q], semi[q])

        def wait_idx(t, q):
            sl = pl.ds(base0 + t * WIN, WIN)
            pltpu.make_async_copy(src_hbm.at[sl], si[q], semi[q]).wait()
            pltpu.make_async_copy(dst_hbm.at[sl], di[q], semi[q]).wait()

        def drain_scatter(k):
            pltpu.make_async_copy(rows[k % 4], acc_sh.at[di[k % 8]],
                                  sems[k % 4]).wait()

        def stage(tb, k):
            t = tb + k

            @pl.when(t >= 4)
            def _():
                drain_scatter(k - 4)

            @pl.when(t + 4 < n_win)
            def _():
                issue_idx(t + 4, (k + 4) % 8)

            @pl.when(t < n_win)
            def _():
                wait_idx(t, k)
                pltpu.async_copy(m_hbm.at[si[k]], rows[k % 4], semg[k % 4])

            @pl.when(jnp.logical_and(t >= 2, t - 2 < n_win))
            def _():
                pltpu.make_async_copy(m_hbm.at[si[(k - 2) % 8]],
                                      rows[(k - 2) % 4],
                                      semg[(k - 2) % 4]).wait()
                pltpu.async_copy(rows[(k - 2) % 4],
                                 acc_sh.at[di[(k - 2) % 8]],
                                 sems[(k - 2) % 4], add=True)

        for t0 in range(4):
            issue_idx(t0, t0)

        @pl.loop(0, n_loop, step=8)
        def _(tb):
            for k in range(8):
                stage(tb, k)

        drain_scatter(n_win - 1)

        plsc.subcore_barrier()
        pltpu.sync_copy(acc_sh.at[pl.ds(sid * rps, rps)],
                        out_hbm.at[cid, pl.ds(sid * rps, rps)])

    return agg


def _split_body(e_ref, s_ref, d_ref):
    v = e_ref[...]
    s_ref[...] = v[0]
    d_ref[...] = v[1]


def _xws_body(x_ref, w_ref, cnt_ref, m_ref, s_ref):
    cb = cnt_ref[...]
    s_row = lax.rsqrt(cb[0:1] + cb[1:2] + 1.0)
    s_col = jnp.transpose(s_row, (1, 0))
    s_ref[...] = s_col
    xw = jnp.dot(x_ref[...], w_ref[...],
                 preferred_element_type=jnp.float32)
    m_ref[...] = xw * s_col


def _gru_body(acc_ref, s_ref, bgcn_ref, wih_ref, bih_ref, bhh_ref,
              wfc_ref, bfc_ref, o_ref, *, hid):
    acc2 = acc_ref[...]
    g = s_ref[...] * (acc2[0] + acc2[1]) + bgcn_ref[...]
    gi = jnp.dot(g.astype(jnp.bfloat16), wih_ref[...],
                 preferred_element_type=jnp.float32)
    gi = gi + bih_ref[...]
    bhh = bhh_ref[...]
    r = jax.nn.sigmoid(gi[:, :hid] + bhh[:, :hid])
    z = jax.nn.sigmoid(gi[:, hid:2 * hid] + bhh[:, hid:2 * hid])
    nn_ = jnp.tanh(gi[:, 2 * hid:] + r * bhh[:, 2 * hid:])
    h = (1.0 - z) * nn_
    o_ref[...] = jnp.dot(h.astype(jnp.bfloat16), wfc_ref[...],
                         preferred_element_type=jnp.float32) + bfc_ref[...]


def kernel(x_seq, edge_idx, W_gcn, b_gcn, W_ih, b_ih, W_hh, b_hh, W_fc, b_fc):
    n, t_in = x_seq.shape
    hid = W_gcn.shape[1]
    e = edge_idx.shape[1]
    n_pad = ((n + NS * WIN - 1) // (NS * WIN)) * (NS * WIN)
    blk = 1000
    grid = (n // blk,)

    blk_e = e
    src, dst = pl.pallas_call(
        _split_body,
        grid=(e // blk_e,),
        in_specs=[pl.BlockSpec((2, blk_e), lambda i: (0, i))],
        out_specs=[pl.BlockSpec((blk_e,), lambda i: (i,)),
                   pl.BlockSpec((blk_e,), lambda i: (i,))],
        out_shape=[jax.ShapeDtypeStruct((e,), jnp.int32),
                   jax.ShapeDtypeStruct((e,), jnp.int32)],
    )(edge_idx)

    cnt = _make_deg(e, n_pad)(dst)

    blk_x = 2560
    m, s_col = pl.pallas_call(
        _xws_body,
        grid=(n_pad // blk_x,),
        in_specs=[pl.BlockSpec((blk_x, t_in), lambda i: (i, 0)),
                  pl.BlockSpec((t_in, hid), lambda i: (0, 0)),
                  pl.BlockSpec((NC, blk_x), lambda i: (0, i))],
        out_specs=[pl.BlockSpec((blk_x, hid), lambda i: (i, 0)),
                   pl.BlockSpec((blk_x, 1), lambda i: (i, 0))],
        out_shape=[jax.ShapeDtypeStruct((n, hid), jnp.float32),
                   jax.ShapeDtypeStruct((n, 1), jnp.float32)],
    )(x_seq, W_gcn, cnt)

    acc = _make_agg(e, n_pad, hid, n)(m, src, dst)

    blk_g = 2000
    out2 = pl.pallas_call(
        functools.partial(_gru_body, hid=hid),
        grid=(n // blk_g,),
        in_specs=[
            pl.BlockSpec((NC, blk_g, hid), lambda i: (0, i, 0)),
            pl.BlockSpec((blk_g, 1), lambda i: (i, 0)),
            pl.BlockSpec((1, hid), lambda i: (0, 0)),
            pl.BlockSpec((hid, 3 * hid), lambda i: (0, 0)),
            pl.BlockSpec((1, 3 * hid), lambda i: (0, 0)),
            pl.BlockSpec((1, 3 * hid), lambda i: (0, 0)),
            pl.BlockSpec((hid, 1), lambda i: (0, 0)),
            pl.BlockSpec((1, 1), lambda i: (0, 0)),
        ],
        out_specs=pl.BlockSpec((blk_g, 1), lambda i: (i, 0)),
        out_shape=jax.ShapeDtypeStruct((n, 1), jnp.float32),
    )(acc, s_col, b_gcn[None, :], W_ih.T.astype(jnp.bfloat16),
      b_ih[None, :], b_hh[None, :], W_fc.T.astype(jnp.bfloat16),
      b_fc[None, :])

    return out2[:, 0]


# speedup vs baseline: 1.0371x; 1.0065x over previous
import functools

import jax
import jax.numpy as jnp
from jax import lax
from jax.experimental import pallas as pl
from jax.experimental.pallas import tpu as pltpu
from jax.experimental.pallas import tpu_sc as plsc

NC, NS = 2, 16
NW = NC * NS
WIN = 80


def _sc_mesh():
    return plsc.VectorSubcoreMesh(core_axis_name="c", subcore_axis_name="s",
                                  num_cores=NC, num_subcores=NS)

def _make_deg(E, n_pad):
    per_w = E // NW
    n_win = per_w // WIN
    rps = n_pad // NS

    @functools.partial(
        pl.kernel,
        out_type=jax.ShapeDtypeStruct((NC, n_pad), jnp.float32),
        mesh=_sc_mesh(),
        scratch_types=[
            [pltpu.VMEM((WIN,), jnp.int32) for _ in range(4)],
            pltpu.VMEM((WIN,), jnp.float32),
            pltpu.VMEM((rps,), jnp.float32),
            pltpu.VMEM_SHARED((n_pad,), jnp.float32),
            [pltpu.SemaphoreType.DMA for _ in range(4)],
            [pltpu.SemaphoreType.DMA for _ in range(2)],
        ],
    )
    def deg(dst_hbm, out_hbm, di, ones_v, z_v, deg_sh, semi, semd):
        cid = lax.axis_index("c")
        sid = lax.axis_index("s")
        wid = sid * NC + cid
        base0 = wid * per_w

        @pl.loop(0, WIN, step=16)
        def _(i):
            ones_v[pl.ds(i, 16)] = jnp.ones((16,), jnp.float32)

        @pl.loop(0, rps, step=16)
        def _(i):
            z_v[pl.ds(i, 16)] = jnp.zeros((16,), jnp.float32)

        pltpu.sync_copy(z_v, deg_sh.at[pl.ds(sid * rps, rps)])
        plsc.subcore_barrier()

        def issue_idx(t, q):
            pltpu.async_copy(dst_hbm.at[pl.ds(base0 + t * WIN, WIN)], di[q],
                             semi[q])

        def window(t, q, p):
            @pl.when(t >= 2)
            def _():
                pltpu.make_async_copy(ones_v, deg_sh.at[di[q]],
                                      semd[p]).wait()
            @pl.when(t + 2 < n_win)
            def _():
                issue_idx(t + 2, (q + 2) % 4)
            pltpu.make_async_copy(dst_hbm.at[pl.ds(base0 + t * WIN, WIN)],
                                  di[q], semi[q]).wait()
            pltpu.async_copy(ones_v, deg_sh.at[di[q]], semd[p], add=True)

        issue_idx(0, 0)
        issue_idx(1, 1)

        @pl.loop(0, n_win - 1, step=4)
        def _(t):
            window(t, 0, 0)
            window(t + 1, 1, 1)
            window(t + 2, 2, 0)
            window(t + 3, 3, 1)

        window(n_win - 1, 0, 0)
        pltpu.make_async_copy(ones_v, deg_sh.at[di[0]], semd[0]).wait()
        pltpu.make_async_copy(ones_v, deg_sh.at[di[1]], semd[1]).wait()

        plsc.subcore_barrier()
        pltpu.sync_copy(deg_sh.at[pl.ds(sid * rps, rps)],
                        out_hbm.at[cid, pl.ds(sid * rps, rps)])

    return deg


def _make_agg(E, n_pad, hid, n_rows):
    per_w = E // NW
    n_win = per_w // WIN
    rps = n_pad // NS

    n_loop = ((n_win + 7) // 8) * 8
    n_fs = n_rows // rps
    rem = n_rows - n_fs * rps

    @functools.partial(
        pl.kernel,
        out_type=jax.ShapeDtypeStruct((NC, n_pad, hid), jnp.float32),
        mesh=_sc_mesh(),
        scratch_types=[
            [pltpu.VMEM((WIN,), jnp.int32) for _ in range(8)],
            [pltpu.VMEM((WIN,), jnp.int32) for _ in range(8)],
            [pltpu.VMEM((WIN, hid), jnp.float32) for _ in range(4)],
            pltpu.VMEM_SHARED((n_pad, hid), jnp.float32),
            [pltpu.SemaphoreType.DMA for _ in range(8)],
            [pltpu.SemaphoreType.DMA for _ in range(4)],
            [pltpu.SemaphoreType.DMA for _ in range(4)],
        ],
    )
    def agg(m_hbm, src_hbm, dst_hbm, out_hbm, si, di, rows, acc_sh, semi,
            semg, sems):
        cid = lax.axis_index("c")
        sid = lax.axis_index("s")
        wid = sid * NC + cid
        base0 = wid * per_w

        @pl.when(cid == 0)
        def _():
            @pl.when(sid < n_fs)
            def _():
                pltpu.sync_copy(m_hbm.at[pl.ds(sid * rps, rps)],
                                acc_sh.at[pl.ds(sid * rps, rps)])
            if rem:
                @pl.when(sid == n_fs)
                def _():
                    pltpu.sync_copy(m_hbm.at[pl.ds(n_fs * rps, rem)],
                                    acc_sh.at[pl.ds(n_fs * rps, rem)])

        @pl.when(cid == 1)
        def _():
            @pl.loop(0, WIN)
            def _(r):
                @pl.loop(0, hid, step=16)
                def _(k):
                    rows[0][r, pl.ds(k, 16)] = jnp.zeros((16,), jnp.float32)

            @pl.loop(0, rps // WIN)
            def _(t):
                pltpu.sync_copy(rows[0],
                                acc_sh.at[pl.ds(sid * rps + t * WIN, WIN)])

        plsc.subcore_barrier()

        def issue_idx(t, q):
            sl = pl.ds(base0 + t * WIN, WIN)
            pltpu.async_copy(src_hbm.at[sl], si[q], semi[q])
            pltpu.async_copy(dst_hbm.at[sl], di[q], semi[q])

        def wait_idx(t, q):
            sl = pl.ds(base0 + t * WIN, WIN)
            pltpu.make_async_copy(src_hbm.at[sl], si[q], semi[q]).wait()
            pltpu.make_async_copy(dst_hbm.at[sl], di[q], semi[q]).wait()

        def drain_scatter(k):
            pltpu.make_async_copy(rows[k % 4], acc_sh.at[di[k % 8]],
                                  sems[k % 4]).wait()

        def stage(tb, k):
            t = tb + k

            @pl.when(t >= 4)
            def _():
                drain_scatter(k - 4)

            @pl.when(t + 4 < n_win)
            def _():
                issue_idx(t + 4, (k + 4) % 8)

            @pl.when(t < n_win)
            def _():
                wait_idx(t, k)
                pltpu.async_copy(m_hbm.at[si[k]], rows[k % 4], semg[k % 4])

            @pl.when(jnp.logical_and(t >= 2, t - 2 < n_win))
            def _():
                pltpu.make_async_copy(m_hbm.at[si[(k - 2) % 8]],
                                      rows[(k - 2) % 4],
                                      semg[(k - 2) % 4]).wait()
                pltpu.async_copy(rows[(k - 2) % 4],
                                 acc_sh.at[di[(k - 2) % 8]],
                                 sems[(k - 2) % 4], add=True)

        for t0 in range(4):
            issue_idx(t0, t0)

        @pl.loop(0, n_loop, step=8)
        def _(tb):
            for k in range(8):
                stage(tb, k)

        drain_scatter(n_win - 1)

        plsc.subcore_barrier()
        pltpu.sync_copy(acc_sh.at[pl.ds(sid * rps, rps)],
                        out_hbm.at[cid, pl.ds(sid * rps, rps)])

    return agg


def _split_body(e_ref, s_ref, d_ref):
    v = e_ref[...]
    s_ref[...] = v[0]
    d_ref[...] = v[1]


def _xws_body(x_ref, w_ref, cnt_ref, m_ref, s_ref):
    cb = cnt_ref[...]
    s_row = lax.rsqrt(cb[0:1] + cb[1:2] + 1.0)
    s_col = jnp.transpose(s_row, (1, 0))
    s_ref[...] = s_col
    xw = jnp.dot(x_ref[...], w_ref[...],
                 preferred_element_type=jnp.float32)
    m_ref[...] = xw * s_col


def _gru_body(acc_ref, s_ref, bgcn_ref, wih_ref, bih_ref, bhh_ref,
              wfc_ref, bfc_ref, o_ref, *, hid):
    acc2 = acc_ref[...]
    g = s_ref[...] * (acc2[0] + acc2[1]) + bgcn_ref[...]
    gi = jnp.dot(g.astype(jnp.bfloat16), wih_ref[...],
                 preferred_element_type=jnp.float32)
    gi = gi + bih_ref[...]
    bhh = bhh_ref[...]
    r = jax.nn.sigmoid(gi[:, :hid] + bhh[:, :hid])
    z = jax.nn.sigmoid(gi[:, hid:2 * hid] + bhh[:, hid:2 * hid])
    nn_ = jnp.tanh(gi[:, 2 * hid:] + r * bhh[:, 2 * hid:])
    h = (1.0 - z) * nn_
    o_ref[...] = jnp.dot(h.astype(jnp.bfloat16), wfc_ref[...],
                         preferred_element_type=jnp.float32) + bfc_ref[...]


def kernel(x_seq, edge_idx, W_gcn, b_gcn, W_ih, b_ih, W_hh, b_hh, W_fc, b_fc):
    n, t_in = x_seq.shape
    hid = W_gcn.shape[1]
    e = edge_idx.shape[1]
    n_pad = ((n + NS * WIN - 1) // (NS * WIN)) * (NS * WIN)
    blk = 1000
    grid = (n // blk,)

    blk_e = e
    src, dst = pl.pallas_call(
        _split_body,
        grid=(e // blk_e,),
        in_specs=[pl.BlockSpec((2, blk_e), lambda i: (0, i))],
        out_specs=[pl.BlockSpec((blk_e,), lambda i: (i,)),
                   pl.BlockSpec((blk_e,), lambda i: (i,))],
        out_shape=[jax.ShapeDtypeStruct((e,), jnp.int32),
                   jax.ShapeDtypeStruct((e,), jnp.int32)],
    )(edge_idx)

    cnt = _make_deg(e, n_pad)(dst)

    blk_x = 2560
    m, s_col = pl.pallas_call(
        _xws_body,
        grid=(n_pad // blk_x,),
        in_specs=[pl.BlockSpec((blk_x, t_in), lambda i: (i, 0)),
                  pl.BlockSpec((t_in, hid), lambda i: (0, 0)),
                  pl.BlockSpec((NC, blk_x), lambda i: (0, i))],
        out_specs=[pl.BlockSpec((blk_x, hid), lambda i: (i, 0)),
                   pl.BlockSpec((blk_x, 1), lambda i: (i, 0))],
        out_shape=[jax.ShapeDtypeStruct((n, hid), jnp.float32),
                   jax.ShapeDtypeStruct((n, 1), jnp.float32)],
    )(x_seq, W_gcn, cnt)

    acc = _make_agg(e, n_pad, hid, n)(m, src, dst)

    blk_g = 5000
    out2 = pl.pallas_call(
        functools.partial(_gru_body, hid=hid),
        grid=(n // blk_g,),
        in_specs=[
            pl.BlockSpec((NC, blk_g, hid), lambda i: (0, i, 0)),
            pl.BlockSpec((blk_g, 1), lambda i: (i, 0)),
            pl.BlockSpec((1, hid), lambda i: (0, 0)),
            pl.BlockSpec((hid, 3 * hid), lambda i: (0, 0)),
            pl.BlockSpec((1, 3 * hid), lambda i: (0, 0)),
            pl.BlockSpec((1, 3 * hid), lambda i: (0, 0)),
            pl.BlockSpec((hid, 1), lambda i: (0, 0)),
            pl.BlockSpec((1, 1), lambda i: (0, 0)),
        ],
        out_specs=pl.BlockSpec((blk_g, 1), lambda i: (i, 0)),
        out_shape=jax.ShapeDtypeStruct((n, 1), jnp.float32),
    )(acc, s_col, b_gcn[None, :], W_ih.T.astype(jnp.bfloat16),
      b_ih[None, :], b_hh[None, :], W_fc.T.astype(jnp.bfloat16),
      b_fc[None, :])

    return out2[:, 0]
